# Initial kernel scaffold; baseline (speedup 1.0000x reference)
import functools

import jax
import jax.numpy as jnp
import numpy as np
from jax import lax
from jax.experimental import pallas as pl
from jax.experimental.pallas import tpu as pltpu
from jax.experimental.pallas import tpu_sc as plsc

_FIELD_DIMS = [100000] * 26
_F = 26
_ED = 16
_B = 4096
_E = 8
_T = 2
_EMB_OUT = (_F + 1) * _ED

_NW = 32
_PER_W = _B * _F // _NW
_CH = 128
_NCH = _PER_W // _CH
_NPAT = 13

_BB = 512


def _offs_pattern():
    offsets = np.concatenate([[0], np.cumsum(_FIELD_DIMS)[:-1]]).astype(np.int32)
    pat = np.array([offsets[q % _F] for q in range(_NPAT * 16)], dtype=np.int32)
    return pat.reshape(_NPAT, 16)


def _sc_gather_body(idx_hbm, offs_hbm, table_hbm, out_hbm, idx_v, offs_v, rows_v, sem):
    wid = lax.axis_index("s") * 2 + lax.axis_index("c")
    pltpu.sync_copy(idx_hbm.at[wid], idx_v)
    pltpu.sync_copy(offs_hbm, offs_v)
    for j in range(_PER_W // 16):
        r = (16 * j) // _CH
        c = (16 * j) % _CH
        p = j % _NPAT
        idx_v[r, pl.ds(c, 16)] = idx_v[r, pl.ds(c, 16)] + offs_v[p, :]
    copies = [
        pltpu.async_copy(table_hbm.at[idx_v.at[k]], rows_v.at[k], sem)
        for k in range(_NCH)
    ]
    for cpy in copies:
        cpy.wait()
    pltpu.sync_copy(rows_v, out_hbm.at[wid])


def _sc_gather(idx3, offs, table):
    mesh = plsc.VectorSubcoreMesh(core_axis_name="c", subcore_axis_name="s")
    return pl.kernel(
        _sc_gather_body,
        out_type=jax.ShapeDtypeStruct((_NW, _NCH, _CH, _ED), jnp.float32),
        mesh=mesh,
        scratch_types=[
            pltpu.VMEM((_NCH, _CH), jnp.int32),
            pltpu.VMEM((_NPAT, 16), jnp.int32),
            pltpu.VMEM((_NCH, _CH, _ED), jnp.float32),
            pltpu.SemaphoreType.DMA,
        ],
    )(idx3, offs, table)


def _tc_dense_body(cat_ref, nx_ref, num_w_ref, num_b_ref, ew1_ref, eb1_ref,
                   ew2_ref, eb2_ref, gw_ref, gb_ref, tw1_ref, tb1_ref,
                   tw2_ref, tb2_ref, tw3_ref, tb3_ref, out_ref):
    numem = jnp.dot(nx_ref[...], num_w_ref[...],
                    preferred_element_type=jnp.float32) + num_b_ref[...]
    emb = jnp.concatenate([cat_ref[...], numem], axis=1)
    feas = []
    for e in range(_E):
        h = jnp.maximum(
            jnp.dot(emb, ew1_ref[e], preferred_element_type=jnp.float32)
            + eb1_ref[e], 0.0)
        f = jnp.maximum(
            jnp.dot(h, ew2_ref[e], preferred_element_type=jnp.float32)
            + eb2_ref[e], 0.0)
        feas.append(f)
    outs = []
    for t in range(_T):
        g = jnp.dot(emb, gw_ref[t], preferred_element_type=jnp.float32) + gb_ref[t]
        g = jnp.exp(g - jnp.max(g, axis=1, keepdims=True))
        g = g / jnp.sum(g, axis=1, keepdims=True)
        tf = feas[0] * g[:, 0:1]
        for e in range(1, _E):
            tf = tf + feas[e] * g[:, e:e + 1]
        th = jnp.maximum(
            jnp.dot(tf, tw1_ref[t], preferred_element_type=jnp.float32)
            + tb1_ref[t], 0.0)
        th = jnp.maximum(
            jnp.dot(th, tw2_ref[t], preferred_element_type=jnp.float32)
            + tb2_ref[t], 0.0)
        o = jnp.dot(th, tw3_ref[t], preferred_element_type=jnp.float32) + tb3_ref[t]
        outs.append(1.0 / (1.0 + jnp.exp(-o)))
    out_ref[...] = jnp.concatenate(outs, axis=1)


def _tc_dense(cat_emb, numerical_x, num_w, num_b, ew1, eb1, ew2, eb2,
              gw, gb, tw1, tb1, tw2, tb2, tw3, tb3):
    def full(arr):
        nd = arr.ndim
        return pl.BlockSpec(arr.shape, lambda i, _n=nd: (0,) * _n)

    grid = (_B // _BB,)
    return pl.pallas_call(
        _tc_dense_body,
        grid=grid,
        in_specs=[
            pl.BlockSpec((_BB, _F * _ED), lambda i: (i, 0)),
            pl.BlockSpec((_BB, numerical_x.shape[1]), lambda i: (i, 0)),
            full(num_w), full(num_b), full(ew1), full(eb1), full(ew2),
            full(eb2), full(gw), full(gb), full(tw1), full(tb1), full(tw2),
            full(tb2), full(tw3), full(tb3),
        ],
        out_specs=pl.BlockSpec((_BB, _T), lambda i: (i, 0)),
        out_shape=jax.ShapeDtypeStruct((_B, _T), jnp.float32),
    )(cat_emb, numerical_x, num_w, num_b, ew1, eb1, ew2, eb2, gw, gb,
      tw1, tb1, tw2, tb2, tw3, tb3)


def kernel(categorical_x, numerical_x, embedding, num_w, num_b, ew1, eb1,
           ew2, eb2, gw, gb, tw1, tb1, tw2, tb2, tw3, tb3):
    idx3 = categorical_x.reshape(_NW, _NCH, _CH)
    offs = jnp.asarray(_offs_pattern())
    rows = _sc_gather(idx3, offs, embedding)
    cat_emb = rows.reshape(_B, _F * _ED)
    return _tc_dense(cat_emb, numerical_x, num_w, num_b, ew1, eb1, ew2, eb2,
                     gw, gb, tw1, tb1, tw2, tb2, tw3, tb3)


# speedup vs baseline: 1.9030x; 1.9030x over previous
import functools

import jax
import jax.numpy as jnp
import numpy as np
from jax import lax
from jax.experimental import pallas as pl
from jax.experimental.pallas import tpu as pltpu
from jax.experimental.pallas import tpu_sc as plsc

_FIELD_DIMS = [100000] * 26
_F = 26
_ED = 16
_B = 4096
_E = 8
_T = 2
_EMB_OUT = (_F + 1) * _ED

_NW = 32
_PER_W = _B * _F // _NW
_CH = 128
_NCH = _PER_W // _CH
_NPAT = 13

_BB = 512


def _offs_pattern():
    offsets = np.concatenate([[0], np.cumsum(_FIELD_DIMS)[:-1]]).astype(np.int32)
    pat = np.array([offsets[q % _F] for q in range(_NPAT * 16)], dtype=np.int32)
    return pat.reshape(_NPAT, 16)


def _sc_gather_body(idx_hbm, offs_hbm, table_hbm, out_hbm, idx_v, offs_v, rows_v, sem):
    wid = lax.axis_index("s") * 2 + lax.axis_index("c")
    pltpu.sync_copy(idx_hbm.at[wid], idx_v)
    pltpu.sync_copy(offs_hbm, offs_v)
    for j in range(_PER_W // 16):
        r = (16 * j) // _CH
        c = (16 * j) % _CH
        p = j % _NPAT
        idx_v[r, pl.ds(c, 16)] = idx_v[r, pl.ds(c, 16)] + offs_v[p, :]
    copies = [
        pltpu.async_copy(table_hbm.at[idx_v.at[k]], rows_v.at[k], sem)
        for k in range(_NCH)
    ]
    for cpy in copies:
        cpy.wait()
    pltpu.sync_copy(rows_v, out_hbm.at[wid])


def _sc_gather(idx3, offs, table):
    mesh = plsc.VectorSubcoreMesh(core_axis_name="c", subcore_axis_name="s")
    return pl.kernel(
        _sc_gather_body,
        out_type=jax.ShapeDtypeStruct((_NW, _NCH, _CH, _ED), jnp.float32),
        mesh=mesh,
        scratch_types=[
            pltpu.VMEM((_NCH, _CH), jnp.int32),
            pltpu.VMEM((_NPAT, 16), jnp.int32),
            pltpu.VMEM((_NCH, _CH, _ED), jnp.float32),
            pltpu.SemaphoreType.DMA,
        ],
        compiler_params=pltpu.CompilerParams(use_tc_tiling_on_sc=False),
    )(idx3, offs, table)


def _tc_dense_body(cat_ref, nx_ref, num_w_ref, num_b_ref, ew1_ref, eb1_ref,
                   ew2_ref, eb2_ref, gw_ref, gb_ref, tw1_ref, tb1_ref,
                   tw2_ref, tb2_ref, tw3_ref, tb3_ref, out_ref):
    numem = jnp.dot(nx_ref[...], num_w_ref[...],
                    preferred_element_type=jnp.float32) + num_b_ref[...]
    emb = jnp.concatenate([cat_ref[...], numem], axis=1)
    feas = []
    for e in range(_E):
        h = jnp.maximum(
            jnp.dot(emb, ew1_ref[e], preferred_element_type=jnp.float32)
            + eb1_ref[e], 0.0)
        f = jnp.maximum(
            jnp.dot(h, ew2_ref[e], preferred_element_type=jnp.float32)
            + eb2_ref[e], 0.0)
        feas.append(f)
    outs = []
    for t in range(_T):
        g = jnp.dot(emb, gw_ref[t], preferred_element_type=jnp.float32) + gb_ref[t]
        g = jnp.exp(g - jnp.max(g, axis=1, keepdims=True))
        g = g / jnp.sum(g, axis=1, keepdims=True)
        tf = feas[0] * g[:, 0:1]
        for e in range(1, _E):
            tf = tf + feas[e] * g[:, e:e + 1]
        th = jnp.maximum(
            jnp.dot(tf, tw1_ref[t], preferred_element_type=jnp.float32)
            + tb1_ref[t], 0.0)
        th = jnp.maximum(
            jnp.dot(th, tw2_ref[t], preferred_element_type=jnp.float32)
            + tb2_ref[t], 0.0)
        o = jnp.dot(th, tw3_ref[t], preferred_element_type=jnp.float32) + tb3_ref[t]
        outs.append(1.0 / (1.0 + jnp.exp(-o)))
    out_ref[...] = jnp.concatenate(outs, axis=1)


def _tc_dense(cat_emb, numerical_x, num_w, num_b, ew1, eb1, ew2, eb2,
              gw, gb, tw1, tb1, tw2, tb2, tw3, tb3):
    def full(arr):
        nd = arr.ndim
        return pl.BlockSpec(arr.shape, lambda i, _n=nd: (0,) * _n)

    grid = (_B // _BB,)
    return pl.pallas_call(
        _tc_dense_body,
        grid=grid,
        in_specs=[
            pl.BlockSpec((_BB, _F * _ED), lambda i: (i, 0)),
            pl.BlockSpec((_BB, numerical_x.shape[1]), lambda i: (i, 0)),
            full(num_w), full(num_b), full(ew1), full(eb1), full(ew2),
            full(eb2), full(gw), full(gb), full(tw1), full(tb1), full(tw2),
            full(tb2), full(tw3), full(tb3),
        ],
        out_specs=pl.BlockSpec((_BB, _T), lambda i: (i, 0)),
        out_shape=jax.ShapeDtypeStruct((_B, _T), jnp.float32),
    )(cat_emb, numerical_x, num_w, num_b, ew1, eb1, ew2, eb2, gw, gb,
      tw1, tb1, tw2, tb2, tw3, tb3)


def kernel(categorical_x, numerical_x, embedding, num_w, num_b, ew1, eb1,
           ew2, eb2, gw, gb, tw1, tb1, tw2, tb2, tw3, tb3):
    idx3 = categorical_x.reshape(_NW, _NCH, _CH)
    offs = jnp.asarray(_offs_pattern())
    rows = _sc_gather(idx3, offs, embedding)
    cat_emb = rows.reshape(_B, _F * _ED)
    return _tc_dense(cat_emb, numerical_x, num_w, num_b, ew1, eb1, ew2, eb2,
                     gw, gb, tw1, tb1, tw2, tb2, tw3, tb3)
